# baseline (speedup 1.0000x reference)
import functools

import jax
import jax.numpy as jnp
from jax import lax
from jax.experimental import pallas as pl
from jax.experimental.pallas import tpu as pltpu
from jax.experimental.pallas import tpu_sc as plsc


def _make_gather(n_tokens, vocab, embed_dim):
    info = plsc.get_sparse_core_info()
    num_workers = info.num_cores * info.num_subcores
    n_per_w = n_tokens // num_workers
    chunk = 16
    nbuf = 10
    n_chunks = n_per_w // chunk
    mesh = plsc.VectorSubcoreMesh(core_axis_name="c", subcore_axis_name="s")

    @functools.partial(
        pl.kernel,
        mesh=mesh,
        out_type=jax.ShapeDtypeStruct((n_tokens, embed_dim), jnp.float32),
        scratch_types=[
            pltpu.VMEM((n_per_w,), jnp.int32),
            pltpu.VMEM((nbuf, chunk, embed_dim), jnp.float32),
            pltpu.SemaphoreType.DMA((nbuf,)),
            pltpu.SemaphoreType.DMA((nbuf,)),
        ],
    )
    def gather_kernel(ids_hbm, table_hbm, out_hbm, idx_v, rows_v, gsem, wsem):
        wid = lax.axis_index("s") * info.num_cores + lax.axis_index("c")
        base = wid * n_per_w
        pltpu.sync_copy(ids_hbm.at[pl.ds(base, n_per_w)], idx_v)

        def gather(j):
            slot = j % nbuf
            return pltpu.async_copy(
                table_hbm.at[idx_v.at[pl.ds(j * chunk, chunk)]],
                rows_v.at[slot],
                gsem.at[slot],
            )

        def write(j):
            slot = j % nbuf
            return pltpu.async_copy(
                rows_v.at[slot],
                out_hbm.at[pl.ds(base + j * chunk, chunk)],
                wsem.at[slot],
            )

        for j in range(nbuf):
            gather(j)

        def body(j, carry):
            slot = lax.rem(j, nbuf)
            row = pl.multiple_of(j * chunk, chunk)
            pltpu.make_async_copy(
                table_hbm.at[idx_v.at[pl.ds(row, chunk)]],
                rows_v.at[slot],
                gsem.at[slot],
            ).wait()
            pltpu.async_copy(
                rows_v.at[slot],
                out_hbm.at[pl.ds(base + row, chunk)],
                wsem.at[slot],
            ).wait()
            nrow = pl.multiple_of((j + nbuf) * chunk, chunk)
            pltpu.async_copy(
                table_hbm.at[idx_v.at[pl.ds(nrow, chunk)]],
                rows_v.at[slot],
                gsem.at[slot],
            )
            return carry

        lax.fori_loop(0, n_chunks - nbuf, body, 0)
        for j in range(n_chunks - nbuf, n_chunks):
            slot = j % nbuf
            pltpu.make_async_copy(
                table_hbm.at[idx_v.at[pl.ds(j * chunk, chunk)]],
                rows_v.at[slot],
                gsem.at[slot],
            ).wait()
            pltpu.async_copy(
                rows_v.at[slot],
                out_hbm.at[pl.ds(base + j * chunk, chunk)],
                wsem.at[slot],
            ).wait()

    return gather_kernel


def kernel(input_ids, table):
    batch, seq = input_ids.shape
    vocab, embed_dim = table.shape
    n_tokens = batch * seq
    flat_ids = input_ids.reshape(n_tokens)
    out = _make_gather(n_tokens, vocab, embed_dim)(flat_ids, table)
    return out.reshape(batch, seq, embed_dim)


# speedup vs baseline: 1.0143x; 1.0143x over previous
import functools

import jax
import jax.numpy as jnp
from jax import lax
from jax.experimental import pallas as pl
from jax.experimental.pallas import tpu as pltpu
from jax.experimental.pallas import tpu_sc as plsc


def _make_gather(n_tokens, vocab, embed_dim):
    info = plsc.get_sparse_core_info()
    num_workers = info.num_cores * info.num_subcores
    n_per_w = n_tokens // num_workers
    chunk = 32
    nbuf = 5
    n_chunks = n_per_w // chunk
    mesh = plsc.VectorSubcoreMesh(core_axis_name="c", subcore_axis_name="s")

    @functools.partial(
        pl.kernel,
        mesh=mesh,
        out_type=jax.ShapeDtypeStruct((n_tokens, embed_dim), jnp.float32),
        scratch_types=[
            pltpu.VMEM((n_per_w,), jnp.int32),
            pltpu.VMEM((nbuf, chunk, embed_dim), jnp.float32),
            pltpu.SemaphoreType.DMA((nbuf,)),
            pltpu.SemaphoreType.DMA((nbuf,)),
        ],
    )
    def gather_kernel(ids_hbm, table_hbm, out_hbm, idx_v, rows_v, gsem, wsem):
        wid = lax.axis_index("s") * info.num_cores + lax.axis_index("c")
        base = wid * n_per_w
        pltpu.sync_copy(ids_hbm.at[pl.ds(base, n_per_w)], idx_v)

        def gather(j):
            slot = j % nbuf
            return pltpu.async_copy(
                table_hbm.at[idx_v.at[pl.ds(j * chunk, chunk)]],
                rows_v.at[slot],
                gsem.at[slot],
            )

        def write(j):
            slot = j % nbuf
            return pltpu.async_copy(
                rows_v.at[slot],
                out_hbm.at[pl.ds(base + j * chunk, chunk)],
                wsem.at[slot],
            )

        for j in range(nbuf):
            gather(j)

        def body(j, carry):
            slot = lax.rem(j, nbuf)
            row = pl.multiple_of(j * chunk, chunk)
            pltpu.make_async_copy(
                table_hbm.at[idx_v.at[pl.ds(row, chunk)]],
                rows_v.at[slot],
                gsem.at[slot],
            ).wait()
            pltpu.async_copy(
                rows_v.at[slot],
                out_hbm.at[pl.ds(base + row, chunk)],
                wsem.at[slot],
            ).wait()
            nrow = pl.multiple_of((j + nbuf) * chunk, chunk)
            pltpu.async_copy(
                table_hbm.at[idx_v.at[pl.ds(nrow, chunk)]],
                rows_v.at[slot],
                gsem.at[slot],
            )
            return carry

        lax.fori_loop(0, n_chunks - nbuf, body, 0)
        for j in range(n_chunks - nbuf, n_chunks):
            slot = j % nbuf
            pltpu.make_async_copy(
                table_hbm.at[idx_v.at[pl.ds(j * chunk, chunk)]],
                rows_v.at[slot],
                gsem.at[slot],
            ).wait()
            pltpu.async_copy(
                rows_v.at[slot],
                out_hbm.at[pl.ds(base + j * chunk, chunk)],
                wsem.at[slot],
            ).wait()

    return gather_kernel


def kernel(input_ids, table):
    batch, seq = input_ids.shape
    vocab, embed_dim = table.shape
    n_tokens = batch * seq
    flat_ids = input_ids.reshape(n_tokens)
    out = _make_gather(n_tokens, vocab, embed_dim)(flat_ids, table)
    return out.reshape(batch, seq, embed_dim)
